# baseline (speedup 1.0000x reference)
import functools

import jax
import jax.numpy as jnp
from jax.experimental import pallas as pl
from jax.experimental.pallas import tpu as pltpu

LEAKY = 0.1
INF = 3e38


def _fps_kernel(x_ref, o_ref, *, npoint, S, B):
    x = x_ref[...]
    N = S * 128
    nidx = (jax.lax.broadcasted_iota(jnp.int32, (B, S, 128), 1) * 128
            + jax.lax.broadcasted_iota(jnp.int32, (B, S, 128), 2))

    def body(i, state):
        dist, far = state
        o_ref[pl.ds(i, 1), :] = far.reshape(1, B)
        mask = (nidx == far)[:, None, :, :]
        coords = jnp.sum(jnp.where(mask, x, 0.0), axis=(2, 3), keepdims=True)
        d0 = (x[:, 0] - coords[:, 0]) ** 2
        d1 = (x[:, 1] - coords[:, 1]) ** 2
        d2 = (x[:, 2] - coords[:, 2]) ** 2
        d = (d0 + d1) + d2
        dist = jnp.minimum(dist, d)
        m = jnp.max(dist, axis=(1, 2), keepdims=True)
        far = jnp.min(jnp.where(dist == m, nidx, N), axis=(1, 2), keepdims=True)
        return (dist, far)

    init = (jnp.full((B, S, 128), 1e10, jnp.float32),
            jnp.zeros((B, 1, 1), jnp.int32))
    jax.lax.fori_loop(0, npoint, body, init)


def fps_pallas(xyz_bcn, npoint):
    B, _, N = xyz_bcn.shape
    S = N // 128
    x = xyz_bcn.reshape(B, 3, S, 128)
    out = pl.pallas_call(
        functools.partial(_fps_kernel, npoint=npoint, S=S, B=B),
        out_shape=jax.ShapeDtypeStruct((npoint, B), jnp.int32),
    )(x)
    return out.T


def _pcf_kernel(q_ref, pt_ref, th_ref, tm_ref, tl_ref, w1_ref, b1_ref,
                w2_ref, b2_ref, w3_ref, b3_ref, wl_ref, bl_ref, o_ref,
                dist_ref, acc_ref, *, K, TQ, N, C):
    q = q_ref[0]
    p = pt_ref[0]
    qn = jnp.sum(q * q, axis=1, keepdims=True)
    pn = jnp.sum(p * p, axis=0, keepdims=True)
    dist_ref[...] = qn + pn - 2.0 * jnp.dot(q, p, preferred_element_type=jnp.float32)
    acc_ref[...] = jnp.zeros((TQ, 8, C), jnp.float32)
    nidx = jax.lax.broadcasted_iota(jnp.int32, (TQ, N), 1)
    qpad = jnp.concatenate([q, jnp.zeros((TQ, C - 3), jnp.float32)], axis=1)

    def body(k, _):
        d = dist_ref[...]
        m = jnp.min(d, axis=1, keepdims=True)
        idx = jnp.min(jnp.where(d == m, nidx, N), axis=1, keepdims=True)
        oh = nidx == idx
        dist_ref[...] = jnp.where(oh, INF, d)
        ohb = oh.astype(jnp.bfloat16)
        raw = ((jnp.dot(ohb, th_ref[0], preferred_element_type=jnp.float32)
                + jnp.dot(ohb, tm_ref[0], preferred_element_type=jnp.float32))
               + jnp.dot(ohb, tl_ref[0], preferred_element_type=jnp.float32))
        npv = raw - qpad
        gx = npv[:, :3]
        h = jnp.maximum(jnp.dot(gx, w1_ref[...], preferred_element_type=jnp.float32) + b1_ref[...], 0.0)
        h = jnp.maximum(jnp.dot(h, w2_ref[...], preferred_element_type=jnp.float32) + b2_ref[...], 0.0)
        wk = jnp.maximum(jnp.dot(h, w3_ref[...], preferred_element_type=jnp.float32) + b3_ref[...], 0.0)
        acc_ref[...] += wk[:, :, None] * npv[:, None, :]
        return 0

    jax.lax.fori_loop(0, K, body, 0)

    O = bl_ref.shape[-1]
    y = jnp.zeros((TQ, O), jnp.float32)
    for w in range(8):
        y = y + jnp.dot(acc_ref[:, w, :], wl_ref[w], preferred_element_type=jnp.float32)
    y = y + bl_ref[...]
    o_ref[0, 0] = jnp.where(y > 0, y, LEAKY * y)


def pointconv_fused(p, new_xyz_nm, xyz_nm, feat_nm, K):
    B, M, _ = new_xyz_nm.shape
    N = xyz_nm.shape[1]
    Cin = feat_nm.shape[2]
    C = 3 + Cin
    O = p["lin"]["W"].shape[0]
    TQ = min(M, 256)
    MT = M // TQ
    pt = jnp.transpose(xyz_nm, (0, 2, 1))
    tab = jnp.concatenate([xyz_nm, feat_nm], axis=2)
    th = tab.astype(jnp.bfloat16)
    tm = (tab - th.astype(jnp.float32)).astype(jnp.bfloat16)
    tl = ((tab - th.astype(jnp.float32)) - tm.astype(jnp.float32)).astype(jnp.bfloat16)
    wn = p["wn"]
    w1 = wn[0]["W"].T; b1 = wn[0]["b"][None, :]
    w2 = wn[1]["W"].T; b2 = wn[1]["b"][None, :]
    w3 = wn[2]["W"].T; b3 = wn[2]["b"][None, :]
    wl = p["lin"]["W"].reshape(O, C, 8).transpose(2, 1, 0)
    bl = p["lin"]["b"][None, :]
    const = lambda b, mt: (0, 0)
    out = pl.pallas_call(
        functools.partial(_pcf_kernel, K=K, TQ=TQ, N=N, C=C),
        grid=(B, MT),
        in_specs=[
            pl.BlockSpec((1, TQ, 3), lambda b, mt: (b, mt, 0)),
            pl.BlockSpec((1, 3, N), lambda b, mt: (b, 0, 0)),
            pl.BlockSpec((1, N, C), lambda b, mt: (b, 0, 0)),
            pl.BlockSpec((1, N, C), lambda b, mt: (b, 0, 0)),
            pl.BlockSpec((1, N, C), lambda b, mt: (b, 0, 0)),
            pl.BlockSpec((3, 8), const),
            pl.BlockSpec((1, 8), const),
            pl.BlockSpec((8, 8), const),
            pl.BlockSpec((1, 8), const),
            pl.BlockSpec((8, 8), const),
            pl.BlockSpec((1, 8), const),
            pl.BlockSpec((8, C, O), lambda b, mt: (0, 0, 0)),
            pl.BlockSpec((1, O), const),
        ],
        out_specs=pl.BlockSpec((1, 1, TQ, O), lambda b, mt: (b, mt, 0, 0)),
        out_shape=jax.ShapeDtypeStruct((B, MT, TQ, O), jnp.float32),
        scratch_shapes=[pltpu.VMEM((TQ, N), jnp.float32),
                        pltpu.VMEM((TQ, 8, C), jnp.float32)],
    )(new_xyz_nm, pt, th, tm, tl, w1, b1, w2, b2, w3, b3, wl, bl)
    return out.reshape(B, M, O)


def _linear_leaky_kernel(x_ref, w_ref, b_ref, o_ref):
    y = jnp.dot(x_ref[...], w_ref[...], preferred_element_type=jnp.float32)
    y = y + b_ref[...]
    o_ref[...] = jnp.where(y > 0, y, LEAKY * y)


def _linear_leaky(x_nm, p):
    B, N, Ci = x_nm.shape
    O = p["W"].shape[0]
    out = pl.pallas_call(
        _linear_leaky_kernel,
        out_shape=jax.ShapeDtypeStruct((B * N, O), jnp.float32),
    )(x_nm.reshape(B * N, Ci), p["W"].T, p["b"][None, :])
    return out.reshape(B, N, O)


def _gather_rows(points, idx):
    return jax.vmap(lambda pts, i: pts[i])(points, idx)


def _t(a):
    return jnp.transpose(a, (0, 2, 1))


def kernel(xyz, color, params):
    xyz_nm = _t(xyz)
    f0a = _linear_leaky(_t(color), params["level0_lift"])
    f0 = pointconv_fused(params["level0"], xyz_nm, xyz_nm, f0a, 32)
    f0_1 = _linear_leaky(f0, params["level0_1"])

    fps1 = fps_pallas(xyz, 2048)
    nx1 = _gather_rows(xyz_nm, fps1)
    f1r = pointconv_fused(params["level1"], nx1, xyz_nm, f0_1, 32)
    f1 = _linear_leaky(f1r, params["level1_0"])
    f1_2 = _linear_leaky(f1, params["level1_1"])

    fps2 = fps_pallas(_t(nx1), 512)
    nx2 = _gather_rows(nx1, fps2)
    f2r = pointconv_fused(params["level2"], nx2, nx1, f1_2, 32)
    f2 = _linear_leaky(f2r, params["level2_0"])
    f2_3 = _linear_leaky(f2, params["level2_1"])

    fps3 = fps_pallas(_t(nx2), 256)
    nx3 = _gather_rows(nx2, fps3)
    f3r = pointconv_fused(params["level3"], nx3, nx2, f2_3, 32)
    f3 = _linear_leaky(f3r, params["level3_0"])
    f3_4 = _linear_leaky(f3, params["level3_1"])

    fps4 = fps_pallas(_t(nx3), 64)
    nx4 = _gather_rows(nx3, fps4)
    f4 = pointconv_fused(params["level4"], nx4, nx3, f3_4, 32)

    return ((xyz, _t(nx1), _t(nx2), _t(nx3), _t(nx4)),
            (_t(f0), _t(f1), _t(f2), _t(f3), _t(f4)),
            (fps1, fps2, fps3, fps4))


# speedup vs baseline: 1.6010x; 1.6010x over previous
import functools

import jax
import jax.numpy as jnp
from jax.experimental import pallas as pl
from jax.experimental.pallas import tpu as pltpu

LEAKY = 0.1
INF = 3e38


def _fps_kernel(x_ref, o_ref, *, npoint, S, B):
    x = x_ref[...]
    N = S * 128
    nidx = (jax.lax.broadcasted_iota(jnp.int32, (B, S, 128), 1) * 128
            + jax.lax.broadcasted_iota(jnp.int32, (B, S, 128), 2))

    def body(i, state):
        dist, far = state
        o_ref[pl.ds(i, 1), :] = far.reshape(1, B)
        mask = (nidx == far)[:, None, :, :]
        coords = jnp.sum(jnp.where(mask, x, 0.0), axis=(2, 3), keepdims=True)
        d0 = (x[:, 0] - coords[:, 0]) ** 2
        d1 = (x[:, 1] - coords[:, 1]) ** 2
        d2 = (x[:, 2] - coords[:, 2]) ** 2
        d = (d0 + d1) + d2
        dist = jnp.minimum(dist, d)
        m = jnp.max(dist, axis=(1, 2), keepdims=True)
        far = jnp.min(jnp.where(dist == m, nidx, N), axis=(1, 2), keepdims=True)
        return (dist, far)

    init = (jnp.full((B, S, 128), 1e10, jnp.float32),
            jnp.zeros((B, 1, 1), jnp.int32))
    jax.lax.fori_loop(0, npoint, body, init)


def fps_pallas(xyz_bcn, npoint):
    B, _, N = xyz_bcn.shape
    S = N // 128
    x = xyz_bcn.reshape(B, 3, S, 128)
    out = pl.pallas_call(
        functools.partial(_fps_kernel, npoint=npoint, S=S, B=B),
        out_shape=jax.ShapeDtypeStruct((npoint, B), jnp.int32),
    )(x)
    return out.T


def _pcf_kernel(q_ref, pt_ref, tab_ref, w1_ref, b1_ref,
                w2_ref, b2_ref, w3_ref, b3_ref, wl_ref, bl_ref, o_ref,
                dist_ref, acc_ref, *, K, TQ, N, C):
    q = q_ref[0]
    p = pt_ref[0]
    qn = jnp.sum(q * q, axis=1, keepdims=True)
    pn = jnp.sum(p * p, axis=0, keepdims=True)
    dist_ref[...] = qn + pn - 2.0 * jnp.dot(q, p, preferred_element_type=jnp.float32)
    acc_ref[...] = jnp.zeros((TQ, 8, C), jnp.float32)
    nidx = jax.lax.broadcasted_iota(jnp.int32, (TQ, N), 1)
    qpad = jnp.concatenate([q, jnp.zeros((TQ, C - 3), jnp.float32)], axis=1)

    def body(k, m):
        d = dist_ref[...]
        eq = d == m
        idx = jnp.min(jnp.where(eq, nidx, N), axis=1, keepdims=True)
        oh = nidx == idx
        dnew = jnp.where(oh, INF, d)
        dist_ref[...] = dnew
        mnew = jnp.min(dnew, axis=1, keepdims=True)
        raw = jnp.dot(oh.astype(jnp.float32), tab_ref[0],
                      preferred_element_type=jnp.float32)
        npv = raw - qpad
        gx = npv[:, :3]
        h = jnp.maximum(jnp.dot(gx, w1_ref[...], preferred_element_type=jnp.float32) + b1_ref[...], 0.0)
        h = jnp.maximum(jnp.dot(h, w2_ref[...], preferred_element_type=jnp.float32) + b2_ref[...], 0.0)
        wk = jnp.maximum(jnp.dot(h, w3_ref[...], preferred_element_type=jnp.float32) + b3_ref[...], 0.0)
        acc_ref[...] += wk[:, :, None] * npv[:, None, :]
        return mnew

    m0 = jnp.min(dist_ref[...], axis=1, keepdims=True)
    jax.lax.fori_loop(0, K, body, m0)

    O = bl_ref.shape[-1]
    y = jnp.zeros((TQ, O), jnp.float32)
    for w in range(8):
        y = y + jnp.dot(acc_ref[:, w, :], wl_ref[w], preferred_element_type=jnp.float32)
    y = y + bl_ref[...]
    o_ref[0, 0] = jnp.where(y > 0, y, LEAKY * y)


def pointconv_fused(p, new_xyz_nm, xyz_nm, feat_nm, K):
    B, M, _ = new_xyz_nm.shape
    N = xyz_nm.shape[1]
    Cin = feat_nm.shape[2]
    C = 3 + Cin
    O = p["lin"]["W"].shape[0]
    TQ = min(M, 512)
    MT = M // TQ
    pt = jnp.transpose(xyz_nm, (0, 2, 1))
    tab = jnp.concatenate([xyz_nm, feat_nm], axis=2)
    wn = p["wn"]
    w1 = wn[0]["W"].T; b1 = wn[0]["b"][None, :]
    w2 = wn[1]["W"].T; b2 = wn[1]["b"][None, :]
    w3 = wn[2]["W"].T; b3 = wn[2]["b"][None, :]
    wl = p["lin"]["W"].reshape(O, C, 8).transpose(2, 1, 0)
    bl = p["lin"]["b"][None, :]
    const = lambda b, mt: (0, 0)
    out = pl.pallas_call(
        functools.partial(_pcf_kernel, K=K, TQ=TQ, N=N, C=C),
        grid=(B, MT),
        in_specs=[
            pl.BlockSpec((1, TQ, 3), lambda b, mt: (b, mt, 0)),
            pl.BlockSpec((1, 3, N), lambda b, mt: (b, 0, 0)),
            pl.BlockSpec((1, N, C), lambda b, mt: (b, 0, 0)),
            pl.BlockSpec((3, 8), const),
            pl.BlockSpec((1, 8), const),
            pl.BlockSpec((8, 8), const),
            pl.BlockSpec((1, 8), const),
            pl.BlockSpec((8, 8), const),
            pl.BlockSpec((1, 8), const),
            pl.BlockSpec((8, C, O), lambda b, mt: (0, 0, 0)),
            pl.BlockSpec((1, O), const),
        ],
        out_specs=pl.BlockSpec((1, 1, TQ, O), lambda b, mt: (b, mt, 0, 0)),
        out_shape=jax.ShapeDtypeStruct((B, MT, TQ, O), jnp.float32),
        scratch_shapes=[pltpu.VMEM((TQ, N), jnp.float32),
                        pltpu.VMEM((TQ, 8, C), jnp.float32)],
    )(new_xyz_nm, pt, tab, w1, b1, w2, b2, w3, b3, wl, bl)
    return out.reshape(B, M, O)


def _linear_leaky_kernel(x_ref, w_ref, b_ref, o_ref):
    y = jnp.dot(x_ref[...], w_ref[...], preferred_element_type=jnp.float32)
    y = y + b_ref[...]
    o_ref[...] = jnp.where(y > 0, y, LEAKY * y)


def _linear_leaky(x_nm, p):
    B, N, Ci = x_nm.shape
    O = p["W"].shape[0]
    out = pl.pallas_call(
        _linear_leaky_kernel,
        out_shape=jax.ShapeDtypeStruct((B * N, O), jnp.float32),
    )(x_nm.reshape(B * N, Ci), p["W"].T, p["b"][None, :])
    return out.reshape(B, N, O)


def _gather_rows(points, idx):
    return jax.vmap(lambda pts, i: pts[i])(points, idx)


def _t(a):
    return jnp.transpose(a, (0, 2, 1))


def kernel(xyz, color, params):
    xyz_nm = _t(xyz)
    f0a = _linear_leaky(_t(color), params["level0_lift"])
    f0 = pointconv_fused(params["level0"], xyz_nm, xyz_nm, f0a, 32)
    f0_1 = _linear_leaky(f0, params["level0_1"])

    fps1 = fps_pallas(xyz, 2048)
    nx1 = _gather_rows(xyz_nm, fps1)
    f1r = pointconv_fused(params["level1"], nx1, xyz_nm, f0_1, 32)
    f1 = _linear_leaky(f1r, params["level1_0"])
    f1_2 = _linear_leaky(f1, params["level1_1"])

    fps2 = fps_pallas(_t(nx1), 512)
    nx2 = _gather_rows(nx1, fps2)
    f2r = pointconv_fused(params["level2"], nx2, nx1, f1_2, 32)
    f2 = _linear_leaky(f2r, params["level2_0"])
    f2_3 = _linear_leaky(f2, params["level2_1"])

    fps3 = fps_pallas(_t(nx2), 256)
    nx3 = _gather_rows(nx2, fps3)
    f3r = pointconv_fused(params["level3"], nx3, nx2, f2_3, 32)
    f3 = _linear_leaky(f3r, params["level3_0"])
    f3_4 = _linear_leaky(f3, params["level3_1"])

    fps4 = fps_pallas(_t(nx3), 64)
    nx4 = _gather_rows(nx3, fps4)
    f4 = pointconv_fused(params["level4"], nx4, nx3, f3_4, 32)

    return ((xyz, _t(nx1), _t(nx2), _t(nx3), _t(nx4)),
            (_t(f0), _t(f1), _t(f2), _t(f3), _t(f4)),
            (fps1, fps2, fps3, fps4))
